# baseline (speedup 1.0000x reference)
import functools

import jax
import jax.numpy as jnp
from jax import lax
from jax.experimental import pallas as pl
from jax.experimental.pallas import tpu as pltpu
from jax.experimental.pallas import tpu_sc as plsc

N_PAD = 10240
F_OUT = 32
PERIODS = 12
NCOLS = 24
NQ = 4
CPT = 3

_MESH = dict(core_axis_name="c", subcore_axis_name="s", num_cores=2,
             num_subcores=16)


def _deg_kernel(e_total):
    per_tile = e_total // 32

    @functools.partial(
        pl.kernel,
        out_type=jax.ShapeDtypeStruct((32, N_PAD), jnp.float32),
        mesh=plsc.VectorSubcoreMesh(**_MESH),
        compiler_params=pltpu.CompilerParams(needs_layout_passes=False),
        scratch_types=[
            pltpu.VMEM((N_PAD,), jnp.float32),
            pltpu.VMEM((per_tile,), jnp.int32),
            pltpu.VMEM((per_tile,), jnp.float32),
        ],
    )
    def deg_kernel(pk_h, w_h, out_h, deg_v, pk_b, w_b):
        wid = lax.axis_index("c") * 16 + lax.axis_index("s")
        zeros = jnp.zeros((16,), jnp.float32)

        @plsc.parallel_loop(0, N_PAD // 16, unroll=8)
        def zero_body(i):
            deg_v[pl.ds(i * 16, 16)] = zeros

        base = wid * per_tile
        pltpu.sync_copy(pk_h.at[pl.ds(base, per_tile)], pk_b)
        pltpu.sync_copy(w_h.at[pl.ds(base, per_tile)], w_b)

        @plsc.parallel_loop(0, per_tile // 16, unroll=8)
        def batch_body(i):
            sl = pl.ds(i * 16, 16)
            d16 = lax.shift_right_logical(pk_b[sl], 16)
            plsc.addupdate_scatter(deg_v, [d16], w_b[sl])

        pltpu.sync_copy(deg_v, out_h.at[wid])

    return deg_kernel


def _pack_kernel(edge_index):
    e_total = edge_index.shape[1]

    def body(ei_ref, out_ref):
        ei = ei_ref[...]
        out_ref[...] = jnp.bitwise_or(
            jnp.left_shift(ei[1:2, :], 16), ei[0:1, :])[0]

    return pl.pallas_call(
        body,
        out_shape=jax.ShapeDtypeStruct((e_total,), jnp.int32),
    )(edge_index)


def _dinv_kernel(degp, xt):
    def body(degp_ref, xt_ref, dinv_ref, xs_ref):
        deg = 1.0 + jnp.sum(degp_ref[...], axis=0, keepdims=True)
        dinv = lax.rsqrt(deg)
        dinv_ref[...] = dinv
        xs_ref[...] = dinv * xt_ref[...]

    return pl.pallas_call(
        body,
        out_shape=[
            jax.ShapeDtypeStruct((1, N_PAD), jnp.float32),
            jax.ShapeDtypeStruct((NCOLS, N_PAD), jnp.float32),
        ],
    )(degp, xt)


def _msg_kernel(e_total):
    chunk = 8000
    e_q = e_total // NQ
    n_chunks = e_q // chunk
    n_pairs = n_chunks // 2

    @functools.partial(
        pl.kernel,
        out_type=jax.ShapeDtypeStruct((NQ * NCOLS, N_PAD), jnp.float32),
        mesh=plsc.VectorSubcoreMesh(**_MESH),
        compiler_params=pltpu.CompilerParams(needs_layout_passes=False),
        scratch_types=[
            pltpu.VMEM((N_PAD,), jnp.float32),
            pltpu.VMEM((N_PAD,), jnp.float32),
            pltpu.VMEM((N_PAD,), jnp.float32),
            pltpu.VMEM((N_PAD,), jnp.float32),
            pltpu.VMEM((N_PAD,), jnp.float32),
            pltpu.VMEM((N_PAD,), jnp.float32),
            pltpu.VMEM((chunk,), jnp.int32),
            pltpu.VMEM((chunk,), jnp.int32),
            pltpu.VMEM((chunk,), jnp.float32),
            pltpu.VMEM((chunk,), jnp.float32),
            pltpu.SemaphoreType.DMA,
            pltpu.SemaphoreType.DMA,
        ],
    )
    def msg_kernel(pk_h, w_h, xs_h, out_h,
                   x0, x1, x2, s0, s1, s2,
                   pk_b0, pk_b1, w_b0, w_b1,
                   sem_a, sem_b):
        wid = lax.axis_index("c") * 16 + lax.axis_index("s")
        q = wid // 8
        cbase = (wid % 8) * CPT

        base = q * e_q
        bufs = ((pk_b0, w_b0), (pk_b1, w_b1))

        def start(j, buf, sem):
            off = base + j * chunk
            pb, wb = bufs[buf]
            pltpu.make_async_copy(
                pk_h.at[pl.ds(off, chunk)], pb, sem).start()
            pltpu.make_async_copy(
                w_h.at[pl.ds(off, chunk)], wb, sem).start()

        def wait(buf, sem):
            pb, wb = bufs[buf]
            pltpu.make_async_copy(
                pk_h.at[pl.ds(base, chunk)], pb, sem).wait()
            pltpu.make_async_copy(
                w_h.at[pl.ds(base, chunk)], wb, sem).wait()

        start(0, 0, sem_a)

        pltpu.sync_copy(xs_h.at[cbase], x0)
        pltpu.sync_copy(xs_h.at[cbase + 1], x1)
        pltpu.sync_copy(xs_h.at[cbase + 2], x2)

        zeros = jnp.zeros((16,), jnp.float32)

        @plsc.parallel_loop(0, N_PAD // 16, unroll=8)
        def zero_body(i):
            sl = pl.ds(i * 16, 16)
            s0[sl] = zeros
            s1[sl] = zeros
            s2[sl] = zeros

        def compute(buf):
            pv, wv = bufs[buf]

            @plsc.parallel_loop(0, chunk // 16, unroll=8)
            def batch_body(i):
                sl = pl.ds(i * 16, 16)
                p16 = pv[sl]
                s16 = jnp.bitwise_and(p16, 0xFFFF)
                d16 = lax.shift_right_logical(p16, 16)
                scale = wv[sl]
                g0 = plsc.load_gather(x0, [s16])
                plsc.addupdate_scatter(s0, [d16], g0 * scale)
                g1 = plsc.load_gather(x1, [s16])
                plsc.addupdate_scatter(s1, [d16], g1 * scale)
                g2 = plsc.load_gather(x2, [s16])
                plsc.addupdate_scatter(s2, [d16], g2 * scale)

        def pair_body(k, _):
            start(2 * k + 1, 1, sem_b)
            wait(0, sem_a)
            compute(0)

            @pl.when(k < n_pairs - 1)
            def _():
                start(2 * k + 2, 0, sem_a)

            wait(1, sem_b)
            compute(1)
            return 0

        lax.fori_loop(0, n_pairs, pair_body, 0)

        row = q * NCOLS + cbase
        pltpu.sync_copy(s0, out_h.at[row])
        pltpu.sync_copy(s1, out_h.at[row + 1])
        pltpu.sync_copy(s2, out_h.at[row + 2])

    return msg_kernel


def _dense_kernel(sp, xt, dinv2d, wzT, bz, whT, bh, probs, wlT, bl):
    bw = 2048
    grid = (N_PAD // bw,)

    def body(sp_ref, xt_ref, dv_ref, wz_ref, bz_ref, wh_ref, bh_ref,
             pr_ref, wl_ref, bl_ref, out_ref):
        spv = sp_ref[...]
        s24 = (spv[0:24] + spv[24:48] + spv[48:72] + spv[72:96])
        d = dv_ref[...]
        y = d * (s24 + xt_ref[...])
        wz = wz_ref[...]
        wh = wh_ref[...]
        bzv = bz_ref[...]
        bhv = bh_ref[...]
        acc = jnp.zeros((F_OUT, bw), jnp.float32)
        for p in range(PERIODS):
            y0 = y[p:p + 1]
            y1 = y[PERIODS + p:PERIODS + p + 1]
            zc = jax.nn.sigmoid(wz[:, 0:1] * y0 + wz[:, 1:2] * y1 + bzv)
            ht = jnp.tanh(wh[:, 0:1] * y0 + wh[:, 1:2] * y1 + bhv)
            acc = acc + pr_ref[0, p] * (zc * ht)
        out_ref[...] = (
            jnp.dot(wl_ref[...], jnp.maximum(acc, 0.0),
                    preferred_element_type=jnp.float32) + bl_ref[...])

    return pl.pallas_call(
        body,
        grid=grid,
        in_specs=[
            pl.BlockSpec((NQ * NCOLS, bw), lambda i: (0, i)),
            pl.BlockSpec((NCOLS, bw), lambda i: (0, i)),
            pl.BlockSpec((1, bw), lambda i: (0, i)),
            pl.BlockSpec((F_OUT, 2), lambda i: (0, 0)),
            pl.BlockSpec((F_OUT, 1), lambda i: (0, 0)),
            pl.BlockSpec((F_OUT, 2), lambda i: (0, 0)),
            pl.BlockSpec((F_OUT, 1), lambda i: (0, 0)),
            pl.BlockSpec(memory_space=pltpu.SMEM),
            pl.BlockSpec((PERIODS, F_OUT), lambda i: (0, 0)),
            pl.BlockSpec((PERIODS, 1), lambda i: (0, 0)),
        ],
        out_specs=pl.BlockSpec((PERIODS, bw), lambda i: (0, i)),
        out_shape=jax.ShapeDtypeStruct((PERIODS, N_PAD), jnp.float32),
    )(sp, xt, dinv2d, wzT, bz, whT, bh, probs, wlT, bl)


def kernel(x, edge_index, edge_weight, W_z, b_z, Wl_z, bl_z, W_r, b_r,
           Wl_r, bl_r, W_h, b_h, Wl_h, bl_h, attention, W_lin, b_lin):
    n = x.shape[0]
    e_total = edge_weight.shape[0]

    xt = jnp.pad(x.transpose(1, 2, 0).reshape(NCOLS, n),
                 ((0, 0), (0, N_PAD - n)))

    packed = _pack_kernel(edge_index)
    degp = _deg_kernel(e_total)(packed, edge_weight)
    dinv2d, xs = _dinv_kernel(degp, xt)
    sp = _msg_kernel(e_total)(packed, edge_weight, xs)

    wz_eff = W_z @ Wl_z[:F_OUT]
    bz_eff = b_z @ Wl_z[:F_OUT] + bl_z
    wh_eff = W_h @ Wl_h[:F_OUT]
    bh_eff = b_h @ Wl_h[:F_OUT] + bl_h
    probs = jax.nn.softmax(attention)

    outT = _dense_kernel(
        sp, xs, dinv2d,
        -wz_eff.T, -bz_eff[:, None], wh_eff.T, bh_eff[:, None],
        probs[None, :], W_lin.T, b_lin[:, None])
    return outT.T[:n]


# speedup vs baseline: 1.0562x; 1.0562x over previous
import functools

import jax
import jax.numpy as jnp
from jax import lax
from jax.experimental import pallas as pl
from jax.experimental.pallas import tpu as pltpu
from jax.experimental.pallas import tpu_sc as plsc

N_PAD = 10240
F_OUT = 32
PERIODS = 12
NCOLS = 24
NQ = 4
CPT = 3

_MESH = dict(core_axis_name="c", subcore_axis_name="s", num_cores=2,
             num_subcores=16)


def _deg_kernel(e_total):
    per_tile = e_total // 32

    @functools.partial(
        pl.kernel,
        out_type=jax.ShapeDtypeStruct((32, N_PAD), jnp.float32),
        mesh=plsc.VectorSubcoreMesh(**_MESH),
        compiler_params=pltpu.CompilerParams(needs_layout_passes=False),
        scratch_types=[
            pltpu.VMEM((N_PAD,), jnp.float32),
            pltpu.VMEM((per_tile,), jnp.int32),
            pltpu.VMEM((per_tile,), jnp.float32),
        ],
    )
    def deg_kernel(pk_h, w_h, out_h, deg_v, pk_b, w_b):
        wid = lax.axis_index("c") * 16 + lax.axis_index("s")
        zeros = jnp.zeros((16,), jnp.float32)

        @plsc.parallel_loop(0, N_PAD // 16, unroll=8)
        def zero_body(i):
            deg_v[pl.ds(i * 16, 16)] = zeros

        base = wid * per_tile
        pltpu.sync_copy(pk_h.at[pl.ds(base, per_tile)], pk_b)
        pltpu.sync_copy(w_h.at[pl.ds(base, per_tile)], w_b)

        @plsc.parallel_loop(0, per_tile // 16, unroll=8)
        def batch_body(i):
            sl = pl.ds(i * 16, 16)
            d16 = lax.shift_right_logical(pk_b[sl], 16)
            plsc.addupdate_scatter(deg_v, [d16], w_b[sl])

        pltpu.sync_copy(deg_v, out_h.at[wid])

    return deg_kernel


def _pack_kernel(edge_index):
    e_total = edge_index.shape[1]

    def body(ei_ref, out_ref):
        ei = ei_ref[...]
        out_ref[...] = jnp.bitwise_or(
            jnp.left_shift(ei[1:2, :], 16), ei[0:1, :])[0]

    return pl.pallas_call(
        body,
        out_shape=jax.ShapeDtypeStruct((e_total,), jnp.int32),
    )(edge_index)


def _dinv_kernel(degp, xt):
    def body(degp_ref, xt_ref, dinv_ref, xs_ref, xp_ref):
        deg = 1.0 + jnp.sum(degp_ref[...], axis=0, keepdims=True)
        dinv = lax.rsqrt(deg)
        dinv_ref[...] = dinv
        xs = dinv * xt_ref[...]
        xs_ref[...] = xs
        xb = xs.astype(jnp.bfloat16)
        lo = lax.bitcast_convert_type(
            xb[:PERIODS], jnp.uint16).astype(jnp.int32)
        hi = lax.bitcast_convert_type(
            xb[PERIODS:], jnp.uint16).astype(jnp.int32)
        xp_ref[...] = jnp.bitwise_or(jnp.left_shift(hi, 16), lo)

    return pl.pallas_call(
        body,
        out_shape=[
            jax.ShapeDtypeStruct((1, N_PAD), jnp.float32),
            jax.ShapeDtypeStruct((NCOLS, N_PAD), jnp.float32),
            jax.ShapeDtypeStruct((PERIODS, N_PAD), jnp.int32),
        ],
    )(degp, xt)


def _msg_kernel(e_total):
    chunk = 4000
    e_8 = e_total // 8
    n_chunks = e_8 // chunk
    n_pairs = n_chunks // 2

    @functools.partial(
        pl.kernel,
        out_type=jax.ShapeDtypeStruct((8 * NCOLS, N_PAD), jnp.float32),
        mesh=plsc.VectorSubcoreMesh(**_MESH),
        compiler_params=pltpu.CompilerParams(needs_layout_passes=False),
        scratch_types=[
            pltpu.VMEM((N_PAD,), jnp.int32),
            pltpu.VMEM((N_PAD,), jnp.int32),
            pltpu.VMEM((N_PAD,), jnp.int32),
            pltpu.VMEM((N_PAD,), jnp.float32),
            pltpu.VMEM((N_PAD,), jnp.float32),
            pltpu.VMEM((N_PAD,), jnp.float32),
            pltpu.VMEM((N_PAD,), jnp.float32),
            pltpu.VMEM((N_PAD,), jnp.float32),
            pltpu.VMEM((N_PAD,), jnp.float32),
            pltpu.VMEM((chunk,), jnp.int32),
            pltpu.VMEM((chunk,), jnp.int32),
            pltpu.VMEM((chunk,), jnp.float32),
            pltpu.VMEM((chunk,), jnp.float32),
            pltpu.SemaphoreType.DMA,
            pltpu.SemaphoreType.DMA,
        ],
    )
    def msg_kernel(pk_h, w_h, xp_h, out_h,
                   x0, x1, x2, sa0, sa1, sa2, sb0, sb1, sb2,
                   pk_b0, pk_b1, w_b0, w_b1,
                   sem_a, sem_b):
        wid = lax.axis_index("c") * 16 + lax.axis_index("s")
        q = wid // 4
        pbase = (wid % 4) * CPT

        base = q * e_8
        bufs = ((pk_b0, w_b0), (pk_b1, w_b1))

        def start(j, buf, sem):
            off = base + j * chunk
            pb, wb = bufs[buf]
            pltpu.make_async_copy(
                pk_h.at[pl.ds(off, chunk)], pb, sem).start()
            pltpu.make_async_copy(
                w_h.at[pl.ds(off, chunk)], wb, sem).start()

        def wait(buf, sem):
            pb, wb = bufs[buf]
            pltpu.make_async_copy(
                pk_h.at[pl.ds(base, chunk)], pb, sem).wait()
            pltpu.make_async_copy(
                w_h.at[pl.ds(base, chunk)], wb, sem).wait()

        start(0, 0, sem_a)

        pltpu.sync_copy(xp_h.at[pbase], x0)
        pltpu.sync_copy(xp_h.at[pbase + 1], x1)
        pltpu.sync_copy(xp_h.at[pbase + 2], x2)

        zeros = jnp.zeros((16,), jnp.float32)

        @plsc.parallel_loop(0, N_PAD // 16, unroll=8)
        def zero_body(i):
            sl = pl.ds(i * 16, 16)
            sa0[sl] = zeros
            sa1[sl] = zeros
            sa2[sl] = zeros
            sb0[sl] = zeros
            sb1[sl] = zeros
            sb2[sl] = zeros

        cols = ((x0, sa0, sb0), (x1, sa1, sb1), (x2, sa2, sb2))

        def compute(buf):
            pv, wv = bufs[buf]

            @plsc.parallel_loop(0, chunk // 16, unroll=8)
            def batch_body(i):
                sl = pl.ds(i * 16, 16)
                p16 = pv[sl]
                s16 = jnp.bitwise_and(p16, 0xFFFF)
                d16 = lax.shift_right_logical(p16, 16)
                scale = wv[sl]
                for xv, sa, sb in cols:
                    g = plsc.load_gather(xv, [s16])
                    a, b = plsc.unpack(
                        plsc.bitcast(g, jnp.bfloat16),
                        format=plsc.PackFormat.INTERLEAVED)
                    plsc.addupdate_scatter(sa, [d16], a * scale)
                    plsc.addupdate_scatter(sb, [d16], b * scale)

        def pair_body(k, _):
            start(2 * k + 1, 1, sem_b)
            wait(0, sem_a)
            compute(0)

            @pl.when(k < n_pairs - 1)
            def _():
                start(2 * k + 2, 0, sem_a)

            wait(1, sem_b)
            compute(1)
            return 0

        lax.fori_loop(0, n_pairs, pair_body, 0)

        row = q * NCOLS + pbase
        pltpu.sync_copy(sa0, out_h.at[row])
        pltpu.sync_copy(sa1, out_h.at[row + 1])
        pltpu.sync_copy(sa2, out_h.at[row + 2])
        pltpu.sync_copy(sb0, out_h.at[row + PERIODS])
        pltpu.sync_copy(sb1, out_h.at[row + PERIODS + 1])
        pltpu.sync_copy(sb2, out_h.at[row + PERIODS + 2])

    return msg_kernel


def _dense_kernel(sp, xt, dinv2d, wzT, bz, whT, bh, probs, wlT, bl):
    bw = 2048
    grid = (N_PAD // bw,)

    def body(sp_ref, xt_ref, dv_ref, wz_ref, bz_ref, wh_ref, bh_ref,
             pr_ref, wl_ref, bl_ref, out_ref):
        spv = sp_ref[...]
        s24 = sum(spv[24 * g:24 * (g + 1)] for g in range(8))
        d = dv_ref[...]
        y = d * (s24 + xt_ref[...])
        wz = wz_ref[...]
        wh = wh_ref[...]
        bzv = bz_ref[...]
        bhv = bh_ref[...]
        acc = jnp.zeros((F_OUT, bw), jnp.float32)
        for p in range(PERIODS):
            y0 = y[p:p + 1]
            y1 = y[PERIODS + p:PERIODS + p + 1]
            zc = jax.nn.sigmoid(wz[:, 0:1] * y0 + wz[:, 1:2] * y1 + bzv)
            ht = jnp.tanh(wh[:, 0:1] * y0 + wh[:, 1:2] * y1 + bhv)
            acc = acc + pr_ref[0, p] * (zc * ht)
        out_ref[...] = (
            jnp.dot(wl_ref[...], jnp.maximum(acc, 0.0),
                    preferred_element_type=jnp.float32) + bl_ref[...])

    return pl.pallas_call(
        body,
        grid=grid,
        in_specs=[
            pl.BlockSpec((8 * NCOLS, bw), lambda i: (0, i)),
            pl.BlockSpec((NCOLS, bw), lambda i: (0, i)),
            pl.BlockSpec((1, bw), lambda i: (0, i)),
            pl.BlockSpec((F_OUT, 2), lambda i: (0, 0)),
            pl.BlockSpec((F_OUT, 1), lambda i: (0, 0)),
            pl.BlockSpec((F_OUT, 2), lambda i: (0, 0)),
            pl.BlockSpec((F_OUT, 1), lambda i: (0, 0)),
            pl.BlockSpec(memory_space=pltpu.SMEM),
            pl.BlockSpec((PERIODS, F_OUT), lambda i: (0, 0)),
            pl.BlockSpec((PERIODS, 1), lambda i: (0, 0)),
        ],
        out_specs=pl.BlockSpec((PERIODS, bw), lambda i: (0, i)),
        out_shape=jax.ShapeDtypeStruct((PERIODS, N_PAD), jnp.float32),
    )(sp, xt, dinv2d, wzT, bz, whT, bh, probs, wlT, bl)


def kernel(x, edge_index, edge_weight, W_z, b_z, Wl_z, bl_z, W_r, b_r,
           Wl_r, bl_r, W_h, b_h, Wl_h, bl_h, attention, W_lin, b_lin):
    n = x.shape[0]
    e_total = edge_weight.shape[0]

    xt = jnp.pad(x.transpose(1, 2, 0).reshape(NCOLS, n),
                 ((0, 0), (0, N_PAD - n)))

    packed = _pack_kernel(edge_index)
    degp = _deg_kernel(e_total)(packed, edge_weight)
    dinv2d, xs, xp = _dinv_kernel(degp, xt)
    sp = _msg_kernel(e_total)(packed, edge_weight, xp)

    wz_eff = W_z @ Wl_z[:F_OUT]
    bz_eff = b_z @ Wl_z[:F_OUT] + bl_z
    wh_eff = W_h @ Wl_h[:F_OUT]
    bh_eff = b_h @ Wl_h[:F_OUT] + bl_h
    probs = jax.nn.softmax(attention)

    outT = _dense_kernel(
        sp, xs, dinv2d,
        -wz_eff.T, -bz_eff[:, None], wh_eff.T, bh_eff[:, None],
        probs[None, :], W_lin.T, b_lin[:, None])
    return outT.T[:n]


# speedup vs baseline: 1.0584x; 1.0021x over previous
import functools

import jax
import jax.numpy as jnp
from jax import lax
from jax.experimental import pallas as pl
from jax.experimental.pallas import tpu as pltpu
from jax.experimental.pallas import tpu_sc as plsc

N_PAD = 10240
F_OUT = 32
PERIODS = 12
NCOLS = 24
NQ = 4
CPT = 3

_MESH = dict(core_axis_name="c", subcore_axis_name="s", num_cores=2,
             num_subcores=16)


def _deg_kernel(e_total):
    per_tile = e_total // 32

    @functools.partial(
        pl.kernel,
        out_type=jax.ShapeDtypeStruct((32, N_PAD), jnp.float32),
        mesh=plsc.VectorSubcoreMesh(**_MESH),
        compiler_params=pltpu.CompilerParams(needs_layout_passes=False),
        scratch_types=[
            pltpu.VMEM((N_PAD,), jnp.float32),
            pltpu.VMEM((per_tile,), jnp.int32),
            pltpu.VMEM((per_tile,), jnp.float32),
        ],
    )
    def deg_kernel(pk_h, w_h, out_h, deg_v, pk_b, w_b):
        wid = lax.axis_index("c") * 16 + lax.axis_index("s")
        zeros = jnp.zeros((16,), jnp.float32)

        @plsc.parallel_loop(0, N_PAD // 16, unroll=8)
        def zero_body(i):
            deg_v[pl.ds(i * 16, 16)] = zeros

        base = wid * per_tile
        pltpu.sync_copy(pk_h.at[pl.ds(base, per_tile)], pk_b)
        pltpu.sync_copy(w_h.at[pl.ds(base, per_tile)], w_b)

        @plsc.parallel_loop(0, per_tile // 16, unroll=8)
        def batch_body(i):
            sl = pl.ds(i * 16, 16)
            d16 = lax.shift_right_logical(pk_b[sl], 16)
            plsc.addupdate_scatter(deg_v, [d16], w_b[sl])

        pltpu.sync_copy(deg_v, out_h.at[wid])

    return deg_kernel


def _pack_kernel(edge_index):
    e_total = edge_index.shape[1]

    def body(ei_ref, out_ref):
        ei = ei_ref[...]
        out_ref[...] = jnp.bitwise_or(
            jnp.left_shift(ei[1:2, :], 16), ei[0:1, :])[0]

    return pl.pallas_call(
        body,
        out_shape=jax.ShapeDtypeStruct((e_total,), jnp.int32),
    )(edge_index)


def _dinv_kernel(degp, xt):
    def body(degp_ref, xt_ref, dinv_ref, xs_ref, xp_ref):
        deg = 1.0 + jnp.sum(degp_ref[...], axis=0, keepdims=True)
        dinv = lax.rsqrt(deg)
        dinv_ref[...] = dinv
        xs = dinv * xt_ref[...]
        xs_ref[...] = xs
        xb = xs.astype(jnp.bfloat16)
        lo = lax.bitcast_convert_type(
            xb[:PERIODS], jnp.uint16).astype(jnp.int32)
        hi = lax.bitcast_convert_type(
            xb[PERIODS:], jnp.uint16).astype(jnp.int32)
        xp_ref[...] = jnp.bitwise_or(jnp.left_shift(hi, 16), lo)

    return pl.pallas_call(
        body,
        out_shape=[
            jax.ShapeDtypeStruct((1, N_PAD), jnp.float32),
            jax.ShapeDtypeStruct((NCOLS, N_PAD), jnp.float32),
            jax.ShapeDtypeStruct((PERIODS, N_PAD), jnp.int32),
        ],
    )(degp, xt)


def _msg_kernel(e_total):
    chunk = 4000
    e_8 = e_total // 8
    n_chunks = e_8 // chunk
    n_pairs = n_chunks // 2

    @functools.partial(
        pl.kernel,
        out_type=jax.ShapeDtypeStruct((8 * NCOLS, N_PAD), jnp.float32),
        mesh=plsc.VectorSubcoreMesh(**_MESH),
        compiler_params=pltpu.CompilerParams(needs_layout_passes=False),
        scratch_types=[
            pltpu.VMEM((N_PAD,), jnp.int32),
            pltpu.VMEM((N_PAD,), jnp.int32),
            pltpu.VMEM((N_PAD,), jnp.int32),
            pltpu.VMEM((N_PAD,), jnp.float32),
            pltpu.VMEM((N_PAD,), jnp.float32),
            pltpu.VMEM((N_PAD,), jnp.float32),
            pltpu.VMEM((N_PAD,), jnp.float32),
            pltpu.VMEM((N_PAD,), jnp.float32),
            pltpu.VMEM((N_PAD,), jnp.float32),
            pltpu.VMEM((chunk,), jnp.int32),
            pltpu.VMEM((chunk,), jnp.int32),
            pltpu.VMEM((chunk,), jnp.float32),
            pltpu.VMEM((chunk,), jnp.float32),
            pltpu.SemaphoreType.DMA,
            pltpu.SemaphoreType.DMA,
        ],
    )
    def msg_kernel(pk_h, w_h, xp_h, out_h,
                   x0, x1, x2, sa0, sa1, sa2, sb0, sb1, sb2,
                   pk_b0, pk_b1, w_b0, w_b1,
                   sem_a, sem_b):
        wid = lax.axis_index("c") * 16 + lax.axis_index("s")
        q = wid // 4
        pbase = (wid % 4) * CPT

        base = q * e_8
        bufs = ((pk_b0, w_b0), (pk_b1, w_b1))

        def start(j, buf, sem):
            off = base + j * chunk
            pb, wb = bufs[buf]
            pltpu.make_async_copy(
                pk_h.at[pl.ds(off, chunk)], pb, sem).start()
            pltpu.make_async_copy(
                w_h.at[pl.ds(off, chunk)], wb, sem).start()

        def wait(buf, sem):
            pb, wb = bufs[buf]
            pltpu.make_async_copy(
                pk_h.at[pl.ds(base, chunk)], pb, sem).wait()
            pltpu.make_async_copy(
                w_h.at[pl.ds(base, chunk)], wb, sem).wait()

        start(0, 0, sem_a)

        pltpu.sync_copy(xp_h.at[pbase], x0)
        pltpu.sync_copy(xp_h.at[pbase + 1], x1)
        pltpu.sync_copy(xp_h.at[pbase + 2], x2)

        zeros = jnp.zeros((16,), jnp.float32)

        @plsc.parallel_loop(0, N_PAD // 16, unroll=8)
        def zero_body(i):
            sl = pl.ds(i * 16, 16)
            sa0[sl] = zeros
            sa1[sl] = zeros
            sa2[sl] = zeros
            sb0[sl] = zeros
            sb1[sl] = zeros
            sb2[sl] = zeros

        cols = ((x0, sa0, sb0), (x1, sa1, sb1), (x2, sa2, sb2))

        def compute(buf):
            pv, wv = bufs[buf]

            @plsc.parallel_loop(0, chunk // 16, unroll=8)
            def batch_body(i):
                sl = pl.ds(i * 16, 16)
                p16 = pv[sl]
                s16 = jnp.bitwise_and(p16, 0xFFFF)
                d16 = lax.shift_right_logical(p16, 16)
                scale = wv[sl]
                for xv, sa, sb in cols:
                    g = plsc.load_gather(xv, [s16])
                    a, b = plsc.unpack(
                        plsc.bitcast(g, jnp.bfloat16),
                        format=plsc.PackFormat.INTERLEAVED)
                    plsc.addupdate_scatter(sa, [d16], a * scale)
                    plsc.addupdate_scatter(sb, [d16], b * scale)

        def pair_body(k, _):
            start(2 * k + 1, 1, sem_b)
            wait(0, sem_a)
            compute(0)

            @pl.when(k < n_pairs - 1)
            def _():
                start(2 * k + 2, 0, sem_a)

            wait(1, sem_b)
            compute(1)
            return 0

        lax.fori_loop(0, n_pairs, pair_body, 0)

        row = q * NCOLS + pbase
        pltpu.sync_copy(sa0, out_h.at[row])
        pltpu.sync_copy(sa1, out_h.at[row + 1])
        pltpu.sync_copy(sa2, out_h.at[row + 2])
        pltpu.sync_copy(sb0, out_h.at[row + PERIODS])
        pltpu.sync_copy(sb1, out_h.at[row + PERIODS + 1])
        pltpu.sync_copy(sb2, out_h.at[row + PERIODS + 2])

    return msg_kernel


def _dense_kernel(sp, xt, dinv2d, wzT, bz, whT, bh, probs, wlT, bl):
    bw = 2048
    grid = (N_PAD // bw,)

    def body(sp_ref, xt_ref, dv_ref, wz_ref, bz_ref, wh_ref, bh_ref,
             pr_ref, wl_ref, bl_ref, out_ref):
        spv = sp_ref[...]
        s24 = sum(spv[24 * g:24 * (g + 1)] for g in range(8))
        d = dv_ref[...]
        y = d * (s24 + xt_ref[...])
        wz = wz_ref[...]
        wh = wh_ref[...]
        bzv = bz_ref[...]
        bhv = bh_ref[...]
        acc = jnp.zeros((F_OUT, bw), jnp.float32)
        for p in range(PERIODS):
            y0 = y[p:p + 1]
            y1 = y[PERIODS + p:PERIODS + p + 1]
            tz = jnp.tanh(wz[:, 0:1] * y0 + wz[:, 1:2] * y1 + bzv)
            ht = jnp.tanh(wh[:, 0:1] * y0 + wh[:, 1:2] * y1 + bhv)
            acc = acc + pr_ref[0, p] * ((1.0 + tz) * ht)
        out_ref[...] = (
            jnp.dot(wl_ref[...], jnp.maximum(acc, 0.0),
                    preferred_element_type=jnp.float32) + bl_ref[...])

    return pl.pallas_call(
        body,
        grid=grid,
        in_specs=[
            pl.BlockSpec((8 * NCOLS, bw), lambda i: (0, i)),
            pl.BlockSpec((NCOLS, bw), lambda i: (0, i)),
            pl.BlockSpec((1, bw), lambda i: (0, i)),
            pl.BlockSpec((F_OUT, 2), lambda i: (0, 0)),
            pl.BlockSpec((F_OUT, 1), lambda i: (0, 0)),
            pl.BlockSpec((F_OUT, 2), lambda i: (0, 0)),
            pl.BlockSpec((F_OUT, 1), lambda i: (0, 0)),
            pl.BlockSpec(memory_space=pltpu.SMEM),
            pl.BlockSpec((PERIODS, F_OUT), lambda i: (0, 0)),
            pl.BlockSpec((PERIODS, 1), lambda i: (0, 0)),
        ],
        out_specs=pl.BlockSpec((PERIODS, bw), lambda i: (0, i)),
        out_shape=jax.ShapeDtypeStruct((PERIODS, N_PAD), jnp.float32),
    )(sp, xt, dinv2d, wzT, bz, whT, bh, probs, wlT, bl)


def kernel(x, edge_index, edge_weight, W_z, b_z, Wl_z, bl_z, W_r, b_r,
           Wl_r, bl_r, W_h, b_h, Wl_h, bl_h, attention, W_lin, b_lin):
    n = x.shape[0]
    e_total = edge_weight.shape[0]

    xt = jnp.pad(x.transpose(1, 2, 0).reshape(NCOLS, n),
                 ((0, 0), (0, N_PAD - n)))

    packed = _pack_kernel(edge_index)
    degp = _deg_kernel(e_total)(packed, edge_weight)
    dinv2d, xs, xp = _dinv_kernel(degp, xt)
    sp = _msg_kernel(e_total)(packed, edge_weight, xp)

    wz_eff = W_z @ Wl_z[:F_OUT]
    bz_eff = b_z @ Wl_z[:F_OUT] + bl_z
    wh_eff = W_h @ Wl_h[:F_OUT]
    bh_eff = b_h @ Wl_h[:F_OUT] + bl_h
    probs = jax.nn.softmax(attention)

    outT = _dense_kernel(
        sp, xs, dinv2d,
        -0.5 * wz_eff.T, -0.5 * bz_eff[:, None], wh_eff.T, bh_eff[:, None],
        0.5 * probs[None, :], W_lin.T, b_lin[:, None])
    return outT.T[:n]
